# baseline (speedup 1.0000x reference)
import functools

import jax
import jax.numpy as jnp
from jax import lax
from jax.experimental import pallas as pl
from jax.experimental.pallas import tpu as pltpu
from jax.experimental.pallas import tpu_sc as plsc

_N = 200
_D = 4
_NP = 256
_NF = _D + 1
_PIDS_NZ = (1, 2, 3, 4)
_BETA2 = 2048.0
_BIG = 3.0e38
_E = 16
_NR = 208
_HALF1 = tuple(range(7))
_HALF2 = tuple(range(7, 13))


def _lane_permute(v, idx):
    dn = lax.GatherDimensionNumbers(offset_dims=(), collapsed_slice_dims=(0,),
                                    start_index_map=(0,))
    return lax.gather(v, idx[:, None], dn, (1,),
                      mode=lax.GatherScatterMode.PROMISE_IN_BOUNDS)


def _bcast_min16(v, lanes):
    for sft in (8, 4, 2, 1):
        v = jnp.minimum(v, _lane_permute(v, jnp.bitwise_xor(lanes, sft)))
    return v


def _sc_min_kernel(fxr_hbm, fy_hbm, rmin_hbm, cmin_hbm, fxr_v, fy_v,
                   rmin_v, cmin_v):
    wid = lax.axis_index("s") * 2 + lax.axis_index("c")
    lanes = lax.iota(jnp.int32, 16)
    for ev in range(2):
        e = wid * 2 + ev
        pltpu.sync_copy(fxr_hbm.at[e], fxr_v)
        pltpu.sync_copy(fy_hbm.at[e], fy_v)

        for half, chunks in ((0, _HALF1), (1, _HALF2)):
            ys = [[fy_v[pl.ds(f * _NP + c * 16, 16)] for f in range(_NF)]
                  for c in chunks]

            def chunk_body(ci, cm, chunks=chunks, ys=ys, half=half):
                base = ci * (16 * _NF * 16)
                rmv = jnp.full((16,), _BIG, jnp.float32)
                for l in range(16):
                    xb = [fxr_v[pl.ds(base + (l * _NF + f) * 16, 16)]
                          for f in range(_NF)]
                    rm = jnp.full((16,), _BIG, jnp.float32)
                    cm2 = []
                    for k, _c in enumerate(chunks):
                        dd = xb[0] - ys[k][0]
                        acc = dd * dd
                        for f in range(1, _NF):
                            dd = xb[f] - ys[k][f]
                            acc = acc + dd * dd
                        rm = jnp.minimum(rm, acc)
                        cm2.append(jnp.minimum(cm[k], acc))
                    cm = tuple(cm2)
                    rmv = jnp.where(lanes == l, _bcast_min16(rm, lanes),
                                    rmv)
                if half == 1:
                    rmv = jnp.minimum(rmv, rmin_v[pl.ds(ci * 16, 16)])
                rmin_v[pl.ds(ci * 16, 16)] = rmv
                return cm

            cm0 = tuple(jnp.full((16,), _BIG, jnp.float32) for _ in chunks)
            cm_fin = lax.fori_loop(0, _NR // 16, chunk_body, cm0,
                                   unroll=False)
            for k, c in enumerate(chunks):
                cmin_v[pl.ds(c * 16, 16)] = cm_fin[k]

        pltpu.sync_copy(rmin_v, rmin_hbm.at[e])
        pltpu.sync_copy(cmin_v, cmin_hbm.at[e])


def _sc_mins(fxr, fy):
    mesh = plsc.VectorSubcoreMesh(core_axis_name="c", subcore_axis_name="s")
    b = fxr.shape[0]
    f32 = jnp.float32
    k = functools.partial(
        pl.kernel, mesh=mesh,
        out_type=[jax.ShapeDtypeStruct((b, _NR), f32),
                  jax.ShapeDtypeStruct((b, _NP), f32)],
        scratch_types=[
            pltpu.VMEM((_NR * _NF * 16,), f32),
            pltpu.VMEM((_NF * _NP,), f32),
            pltpu.VMEM((_NR,), f32),
            pltpu.VMEM((_NP,), f32),
        ],
    )(_sc_min_kernel)
    return k(fxr, fy)


def _tc_assembly(rmin2_ref, cmin2_ref, nx2_ref, ny2_ref, ip2_ref, op2_ref,
                 nz_ref, z_ref, *, n_batches):
    i = pl.program_id(0)

    rmin2 = rmin2_ref[...][:, :_N]
    cmin2 = cmin2_ref[...][:, :_N]
    nx2 = nx2_ref[...]
    ny2 = ny2_ref[...]
    ip2 = ip2_ref[...]
    op2 = op2_ref[...]

    inv_b = jnp.float32(1.0 / n_batches)
    norm_x = jnp.sqrt(nx2)
    norm_y = jnp.sqrt(ny2)
    rmin = jnp.sqrt(jnp.maximum(rmin2, 0.0))
    cmin = jnp.sqrt(jnp.maximum(cmin2, 0.0))

    mz = op2 == 0
    n0 = jnp.maximum(1, jnp.sum(mz, axis=1, keepdims=True)).astype(jnp.float32)
    loss_zero = jnp.sum(jnp.where(mz, norm_y, 0.0), axis=1, keepdims=True) / n0
    z_ref[...] = loss_zero * inv_b

    loss_nz = jnp.zeros((_E, 1), jnp.float32)
    for p in _PIDS_NZ:
        mx = ip2 == p
        my = op2 == p
        nin = jnp.sum(mx, axis=1, keepdims=True)
        nout = jnp.sum(my, axis=1, keepdims=True)
        ninp = jnp.maximum(1, nin).astype(jnp.float32)
        noutp = jnp.maximum(1, nout).astype(jnp.float32)
        s_a = jnp.sum(jnp.where(mx, norm_x, 0.0), axis=1, keepdims=True)
        s_b = jnp.sum(jnp.where(my, norm_y, 0.0), axis=1, keepdims=True)
        s_cx = jnp.sum(jnp.where(mx, rmin, 0.0), axis=1, keepdims=True)
        s_cy = jnp.sum(jnp.where(my, cmin, 0.0), axis=1, keepdims=True)
        loss_a = s_a / ninp
        loss_b = s_b / noutp
        loss_c = 0.5 * (s_cx / noutp + s_cy / ninp)
        loss_p = jnp.where(nout == 0, loss_a, jnp.where(nin == 0, loss_b, loss_c))
        loss_nz = loss_nz + loss_p

    @pl.when(i == 0)
    def _():
        nz_ref[...] = jnp.zeros((1, 1), jnp.float32)

    nz_ref[...] += jnp.sum(loss_nz).reshape(1, 1) * inv_b


def kernel(target, reco, in_pid, out_pid):
    b, n, d = target.shape
    f32 = jnp.float32
    beta = jnp.sqrt(jnp.float32(_BETA2))
    nx2 = jnp.sum(target * target, axis=2)
    ny2 = jnp.sum(reco * reco, axis=2)

    xt = jnp.transpose(target, (0, 2, 1))
    yt = jnp.transpose(reco, (0, 2, 1))
    fx = jnp.concatenate([xt, (beta * in_pid.astype(f32))[:, None, :]], axis=1)
    fy = jnp.concatenate([yt, (beta * out_pid.astype(f32))[:, None, :]], axis=1)
    fxp = jnp.pad(fx, ((0, 0), (0, 0), (0, _NR - n)), constant_values=-1e4)
    fxr = jnp.broadcast_to(
        jnp.transpose(fxp, (0, 2, 1))[:, :, :, None], (b, _NR, _NF, 16)
    ).reshape(b, _NR * _NF * 16)
    fy = jnp.pad(fy, ((0, 0), (0, 0), (0, _NP - n)), constant_values=1e4)
    fy = fy.reshape(b, _NF * _NP)

    rmin2, cmin2 = _sc_mins(fxr, fy)

    steps = b // _E
    nz, z = pl.pallas_call(
        functools.partial(_tc_assembly, n_batches=b),
        grid=(steps,),
        in_specs=[
            pl.BlockSpec((_E, _NR), lambda i: (i, 0)),
            pl.BlockSpec((_E, _NP), lambda i: (i, 0)),
            pl.BlockSpec((_E, n), lambda i: (i, 0)),
            pl.BlockSpec((_E, n), lambda i: (i, 0)),
            pl.BlockSpec((_E, n), lambda i: (i, 0)),
            pl.BlockSpec((_E, n), lambda i: (i, 0)),
        ],
        out_specs=[
            pl.BlockSpec((1, 1), lambda i: (0, 0)),
            pl.BlockSpec((_E, 1), lambda i: (i, 0)),
        ],
        out_shape=[
            jax.ShapeDtypeStruct((1, 1), jnp.float32),
            jax.ShapeDtypeStruct((b, 1), jnp.float32),
        ],
    )(rmin2, cmin2, nx2, ny2, in_pid, out_pid)

    return nz.reshape(()), z.reshape(b)


# speedup vs baseline: 3.1914x; 3.1914x over previous
import functools

import jax
import jax.numpy as jnp
from jax.experimental import pallas as pl

_N = 200
_D = 4
_E = 16
_PIDS_NZ = (1, 2, 3, 4)
_ALPHA = 45.25
_PEN = 2.0 * _ALPHA * _ALPHA
_K = _D + 4


def _chamfer_kernel(lhs_ref, rht_ref, nx2_ref, ny2p_ref, ip2_ref, op2_ref,
                    nz_ref, z_ref, *, n_batches):
    i = pl.program_id(0)

    lhs = lhs_ref[...]
    rht = rht_ref[...]
    nx2 = nx2_ref[...]
    ny2p = ny2p_ref[...]
    ip2 = ip2_ref[...]
    op2 = op2_ref[...]

    inv_b = jnp.float32(1.0 / n_batches)

    norm_x = jnp.sqrt(nx2)
    norm_y = jnp.sqrt(jnp.maximum(ny2p - _PEN, 0.0))

    mz = op2 == 0
    n0 = jnp.maximum(1, jnp.sum(mz, axis=1, keepdims=True)).astype(jnp.float32)
    loss_zero = jnp.sum(jnp.where(mz, norm_y, 0.0), axis=1, keepdims=True) / n0
    z_ref[...] = loss_zero * inv_b

    xy = jax.lax.dot_general(
        lhs, rht, (((2,), (1,)), ((0,), (0,))),
        preferred_element_type=jnp.float32,
    )
    nx2_col = jnp.sum(lhs[:, :, :_D] * lhs[:, :, :_D], axis=2,
                      keepdims=True)
    dm2 = (nx2_col + ny2p[:, None, :]) - 2.0 * xy

    rmin = jnp.sqrt(jnp.maximum(jnp.min(dm2, axis=2), 0.0))
    cmin = jnp.sqrt(jnp.maximum(jnp.min(dm2, axis=1), 0.0))

    loss_nz = jnp.zeros((_E, 1), jnp.float32)
    for p in _PIDS_NZ:
        mx = ip2 == p
        my = op2 == p
        nin = jnp.sum(mx, axis=1, keepdims=True)
        nout = jnp.sum(my, axis=1, keepdims=True)
        ninp = jnp.maximum(1, nin).astype(jnp.float32)
        noutp = jnp.maximum(1, nout).astype(jnp.float32)
        s_a = jnp.sum(jnp.where(mx, norm_x, 0.0), axis=1, keepdims=True)
        s_b = jnp.sum(jnp.where(my, norm_y, 0.0), axis=1, keepdims=True)
        s_cx = jnp.sum(jnp.where(mx, rmin, 0.0), axis=1, keepdims=True)
        s_cy = jnp.sum(jnp.where(my, cmin, 0.0), axis=1, keepdims=True)
        loss_a = s_a / ninp
        loss_b = s_b / noutp
        loss_c = 0.5 * (s_cx / noutp + s_cy / ninp)
        loss_p = jnp.where(nout == 0, loss_a, jnp.where(nin == 0, loss_b, loss_c))
        loss_nz = loss_nz + loss_p

    @pl.when(i == 0)
    def _():
        nz_ref[...] = jnp.zeros((1, 1), jnp.float32)

    nz_ref[...] += jnp.sum(loss_nz).reshape(1, 1) * inv_b


def kernel(target, reco, in_pid, out_pid):
    b, n, d = target.shape
    f32 = jnp.float32
    nx2 = jnp.sum(target * target, axis=2)
    ny2p = jnp.sum(reco * reco, axis=2) + _PEN
    ohx = _ALPHA * jax.nn.one_hot(in_pid - 1, 4, dtype=f32)
    ohy = _ALPHA * jax.nn.one_hot(out_pid - 1, 4, dtype=f32)
    lhs = jnp.concatenate([target, ohx], axis=2)
    rht = jnp.transpose(jnp.concatenate([reco, ohy], axis=2), (0, 2, 1))
    steps = b // _E

    nz, z = pl.pallas_call(
        functools.partial(_chamfer_kernel, n_batches=b),
        grid=(steps,),
        in_specs=[
            pl.BlockSpec((_E, n, _K), lambda i: (i, 0, 0)),
            pl.BlockSpec((_E, _K, n), lambda i: (i, 0, 0)),
            pl.BlockSpec((_E, n), lambda i: (i, 0)),
            pl.BlockSpec((_E, n), lambda i: (i, 0)),
            pl.BlockSpec((_E, n), lambda i: (i, 0)),
            pl.BlockSpec((_E, n), lambda i: (i, 0)),
        ],
        out_specs=[
            pl.BlockSpec((1, 1), lambda i: (0, 0)),
            pl.BlockSpec((_E, 1), lambda i: (i, 0)),
        ],
        out_shape=[
            jax.ShapeDtypeStruct((1, 1), jnp.float32),
            jax.ShapeDtypeStruct((b, 1), jnp.float32),
        ],
    )(lhs, rht, nx2, ny2p, in_pid, out_pid)

    return nz.reshape(()), z.reshape(b)
